# baseline (speedup 1.0000x reference)
import functools

import jax
import jax.numpy as jnp
from jax.experimental import pallas as pl
from jax.experimental.pallas import tpu as pltpu


def _dot(a, b):
    return jax.lax.dot_general(a, b, (((1,), (0,)), ((), ())),
                               preferred_element_type=jnp.float32)


def _fused_kernel(x_ref, wc_ref, bc_ref, w2_ref, b2_ref, w3_ref, b3_ref,
                  wv2_ref, bv2_ref, logits_ref, values_ref, *, d1):
    x = x_ref[...].astype(jnp.bfloat16)
    hc = jnp.maximum(_dot(x, wc_ref[...]).astype(jnp.bfloat16) + bc_ref[...],
                     0)
    h2 = jnp.maximum(
        _dot(hc[:, :d1], w2_ref[...]).astype(jnp.bfloat16) + b2_ref[...], 0)
    logits_ref[...] = _dot(h2, w3_ref[...]) + b3_ref[...]
    values_ref[...] = _dot(hc[:, d1:], wv2_ref[...]) + bv2_ref[...]


def kernel(hidden_states, W1, b1, W2, b2, W3, b3, Wv1, bv1, Wv2, bv2):
    B, S, H = hidden_states.shape
    N = B * S
    d1 = W1.shape[1]
    d2 = W2.shape[1]
    ne = W3.shape[1]

    flat = hidden_states.reshape(N, H)
    bf = jnp.bfloat16
    Wc = jnp.concatenate([W1, Wv1], axis=1).astype(bf)
    bc = jnp.concatenate([b1, bv1]).reshape(1, -1).astype(bf)

    TILE = 4096
    grid = (N // TILE,)

    body = functools.partial(_fused_kernel, d1=d1)

    logits, values = pl.pallas_call(
        body,
        grid=grid,
        in_specs=[
            pl.BlockSpec((TILE, H), lambda i: (i, 0)),
            pl.BlockSpec((H, 2 * d1), lambda i: (0, 0)),
            pl.BlockSpec((1, 2 * d1), lambda i: (0, 0)),
            pl.BlockSpec((d1, d2), lambda i: (0, 0)),
            pl.BlockSpec((1, d2), lambda i: (0, 0)),
            pl.BlockSpec((d2, ne), lambda i: (0, 0)),
            pl.BlockSpec((1, ne), lambda i: (0, 0)),
            pl.BlockSpec((d1, 1), lambda i: (0, 0)),
            pl.BlockSpec((1, 1), lambda i: (0, 0)),
        ],
        out_specs=[
            pl.BlockSpec((TILE, ne), lambda i: (i, 0)),
            pl.BlockSpec((TILE, 1), lambda i: (i, 0)),
        ],
        out_shape=[
            jax.ShapeDtypeStruct((N, ne), jnp.float32),
            jax.ShapeDtypeStruct((N, 1), jnp.float32),
        ],
        compiler_params=pltpu.CompilerParams(
            dimension_semantics=("parallel",),
        ),
    )(flat, Wc, bc, W2.astype(bf), b2.reshape(1, -1).astype(bf),
      W3.astype(bf), b3.reshape(1, -1),
      Wv2.astype(bf), bv2.reshape(1, -1))

    return (logits.reshape(B, S, ne), values.reshape(B, S, 1))


# speedup vs baseline: 1.0092x; 1.0092x over previous
import functools

import jax
import jax.numpy as jnp
from jax.experimental import pallas as pl
from jax.experimental.pallas import tpu as pltpu


def _dot(a, b):
    return jax.lax.dot_general(a, b, (((1,), (0,)), ((), ())),
                               preferred_element_type=jnp.float32)


def _fused_kernel(x_ref, wc_ref, bc_ref, wd_ref, bd_ref, w3_ref, b3_ref,
                  logits_ref, values_ref, *, d2):
    x = x_ref[...].astype(jnp.bfloat16)
    hc = jnp.maximum(_dot(x, wc_ref[...]).astype(jnp.bfloat16) + bc_ref[...],
                     0)
    t = _dot(hc, wd_ref[...]) + bd_ref[...]
    h2 = jnp.maximum(t[:, :d2], 0).astype(jnp.bfloat16)
    values_ref[...] = t[:, d2:d2 + 1]
    logits_ref[...] = _dot(h2, w3_ref[...]) + b3_ref[...]


def kernel(hidden_states, W1, b1, W2, b2, W3, b3, Wv1, bv1, Wv2, bv2):
    B, S, H = hidden_states.shape
    N = B * S
    d1 = W1.shape[1]
    d2 = W2.shape[1]
    ne = W3.shape[1]

    flat = hidden_states.reshape(N, H)
    bf = jnp.bfloat16
    Wc = jnp.concatenate([W1, Wv1], axis=1).astype(bf)
    bc = jnp.concatenate([b1, bv1]).reshape(1, -1).astype(bf)
    Wd = jnp.zeros((2 * d1, d2 + 1), jnp.float32)
    Wd = Wd.at[:d1, :d2].set(W2).at[d1:, d2:].set(Wv2).astype(bf)
    bd = jnp.concatenate([b2, bv2]).reshape(1, -1)

    TILE = 4096
    grid = (N // TILE,)

    body = functools.partial(_fused_kernel, d2=d2)

    logits, values = pl.pallas_call(
        body,
        grid=grid,
        in_specs=[
            pl.BlockSpec((TILE, H), lambda i: (i, 0)),
            pl.BlockSpec((H, 2 * d1), lambda i: (0, 0)),
            pl.BlockSpec((1, 2 * d1), lambda i: (0, 0)),
            pl.BlockSpec((2 * d1, d2 + 1), lambda i: (0, 0)),
            pl.BlockSpec((1, d2 + 1), lambda i: (0, 0)),
            pl.BlockSpec((d2, ne), lambda i: (0, 0)),
            pl.BlockSpec((1, ne), lambda i: (0, 0)),
        ],
        out_specs=[
            pl.BlockSpec((TILE, ne), lambda i: (i, 0)),
            pl.BlockSpec((TILE, 1), lambda i: (i, 0)),
        ],
        out_shape=[
            jax.ShapeDtypeStruct((N, ne), jnp.float32),
            jax.ShapeDtypeStruct((N, 1), jnp.float32),
        ],
        compiler_params=pltpu.CompilerParams(
            dimension_semantics=("arbitrary",),
        ),
    )(flat, Wc, bc, Wd, bd, W3.astype(bf), b3.reshape(1, -1))

    return (logits.reshape(B, S, ne), values.reshape(B, S, 1))
